# baseline (speedup 1.0000x reference)
import functools

import jax
import jax.numpy as jnp
from jax import lax
from jax.experimental import pallas as pl
from jax.experimental.pallas import tpu as pltpu
from jax.experimental.pallas import tpu_sc as plsc

N = 8192
K = 16


ROWS = 512
CT = 4096
NT = N // CT
BIG = 1e30


def _knn_body(pos_ref, post_ref, idx_ref, d_ref):
    pos = pos_ref[...]
    px = pos[:, 0:1]
    py = pos[:, 1:2]
    pz = pos[:, 2:3]
    sqi = px * px + py * py + pz * pz

    def bf(v):
        return v.astype(jnp.bfloat16).astype(jnp.float32)

    pxb, pyb, pzb = bf(px), bf(py), bf(pz)
    lt0 = lax.broadcasted_iota(jnp.int32, (1, CT), 1)

    def fill(ct, _):
        col0 = pl.multiple_of(ct * CT, CT)
        tx = post_ref[0:1, pl.ds(col0, CT)]
        ty = post_ref[1:2, pl.ds(col0, CT)]
        tz = post_ref[2:3, pl.ds(col0, CT)]
        sqj = tx * tx + ty * ty + tz * tz
        dot = pxb * bf(tx) + pyb * bf(ty) + pzb * bf(tz)
        d_ref[:, pl.ds(col0, CT)] = sqi + sqj - 2.0 * dot
        return 0

    lax.fori_loop(0, NT, fill, 0)

    am = jnp.full((ROWS, 1), -1, jnp.int32)
    for t in range(K):
        def tile_body(ct, carry, am=am):
            bm, ba = carry
            col0 = pl.multiple_of(ct * CT, CT)
            lt = lt0 + ct * CT
            dt = d_ref[:, pl.ds(col0, CT)]
            dt = jnp.where(lt == am, BIG, dt)
            d_ref[:, pl.ds(col0, CT)] = dt
            tmin = jnp.min(dt, axis=1, keepdims=True)
            targ = jnp.min(jnp.where(dt == tmin, lt, jnp.int32(N)),
                           axis=1, keepdims=True)
            upd = tmin < bm
            return (jnp.where(upd, tmin, bm), jnp.where(upd, targ, ba))

        bm, am = lax.fori_loop(
            0, NT, tile_body,
            (jnp.full((ROWS, 1), BIG, jnp.float32),
             jnp.zeros((ROWS, 1), jnp.int32)))
        idx_ref[:, t:t + 1] = am


def _knn(pos, post):
    return pl.pallas_call(
        _knn_body,
        grid=(N // ROWS,),
        in_specs=[
            pl.BlockSpec((ROWS, 3), lambda i: (i, 0)),
            pl.BlockSpec((3, N), lambda i: (0, 0)),
        ],
        out_specs=pl.BlockSpec((ROWS, K), lambda i: (i, 0)),
        out_shape=jax.ShapeDtypeStruct((N, K), jnp.int32),
        scratch_shapes=[pltpu.VMEM((ROWS, N), jnp.float32)],
        compiler_params=pltpu.CompilerParams(
            dimension_semantics=("arbitrary",)),
    )(pos, post)


MROWS = 1024


def _mm_body(x_ref, w_ref, b_ref, a_ref, q_ref, *, cin):
    x = x_ref[...]
    wa = w_ref[0:cin, :]
    wb = w_ref[cin:2 * cin, :]
    q = jnp.dot(x, wb, preferred_element_type=jnp.float32,
                precision=lax.Precision.HIGHEST)
    a = (jnp.dot(x, wa, preferred_element_type=jnp.float32,
                 precision=lax.Precision.HIGHEST) - q + b_ref[...])
    a_ref[...] = a
    q_ref[...] = q


def _mm(x, w, b2d):
    cin = w.shape[0] // 2
    cout = w.shape[1]
    return pl.pallas_call(
        functools.partial(_mm_body, cin=cin),
        grid=(N // MROWS,),
        in_specs=[
            pl.BlockSpec((MROWS, cin), lambda i: (i, 0)),
            pl.BlockSpec((2 * cin, cout), lambda i: (0, 0)),
            pl.BlockSpec((1, cout), lambda i: (0, 0)),
        ],
        out_specs=[
            pl.BlockSpec((MROWS, cout), lambda i: (i, 0)),
            pl.BlockSpec((MROWS, cout), lambda i: (i, 0)),
        ],
        out_shape=[jax.ShapeDtypeStruct((N, cout), jnp.float32)] * 2,
        compiler_params=pltpu.CompilerParams(
            dimension_semantics=("parallel",)),
    )(x, w, b2d)


def _sc_reduce(q, idx_flat, C):
    mesh = plsc.VectorSubcoreMesh(core_axis_name="c", subcore_axis_name="s")
    NW = 32
    PW = N // NW
    PC = 8
    NCH = PW // PC
    GR = PC * K
    NV = C // 16

    @functools.partial(
        pl.kernel, mesh=mesh,
        out_type=[jax.ShapeDtypeStruct((N, C), jnp.float32)] * 4,
        scratch_types=[
            pltpu.VMEM((GR,), jnp.int32),
            pltpu.VMEM((GR,), jnp.int32),
            pltpu.VMEM((GR, C), jnp.float32),
            pltpu.VMEM((GR, C), jnp.float32),
            pltpu.VMEM((PC, C), jnp.float32),
            pltpu.VMEM((PC, C), jnp.float32),
            pltpu.VMEM((PC, C), jnp.float32),
            pltpu.VMEM((PC, C), jnp.float32),
            pltpu.SemaphoreType.DMA,
            pltpu.SemaphoreType.DMA,
        ],
    )
    def body(q_hbm, idx_hbm, g_hbm, g2_hbm, mx_hbm, mn_hbm,
             idxA, idxB, rowsA, rowsB, o_sum, o_sq, o_mx, o_mn,
             semA, semB):
        wid = lax.axis_index("s") * 2 + lax.axis_index("c")
        base = wid * PW

        def fire(cidx, idx_v, rows_v, sem):
            off = (base + cidx * PC) * K
            pltpu.sync_copy(idx_hbm.at[pl.ds(off, GR)], idx_v)
            return pltpu.async_copy(q_hbm.at[idx_v], rows_v, sem)

        def reduce_chunk(cidx, rows_v):
            def col_body(t, _):
                p = t // NV
                v = t - p * NV
                col = pl.multiple_of(v * 16, 16)
                r0 = p * K
                x0 = rows_v[r0, pl.ds(col, 16)]
                s = x0
                sq = x0 * x0
                mx = x0
                mn = x0
                for j in range(1, K):
                    x = rows_v[r0 + j, pl.ds(col, 16)]
                    s = s + x
                    sq = sq + x * x
                    mx = jnp.maximum(mx, x)
                    mn = jnp.minimum(mn, x)
                o_sum[p, pl.ds(col, 16)] = s
                o_sq[p, pl.ds(col, 16)] = sq
                o_mx[p, pl.ds(col, 16)] = mx
                o_mn[p, pl.ds(col, 16)] = mn
                return 0

            lax.fori_loop(0, PC * NV, col_body, 0)
            row0 = base + cidx * PC
            pltpu.sync_copy(o_sum, g_hbm.at[pl.ds(row0, PC)])
            pltpu.sync_copy(o_sq, g2_hbm.at[pl.ds(row0, PC)])
            pltpu.sync_copy(o_mx, mx_hbm.at[pl.ds(row0, PC)])
            pltpu.sync_copy(o_mn, mn_hbm.at[pl.ds(row0, PC)])

        def drain(rows_v, sem):
            pltpu.make_async_copy(q_hbm.at[pl.ds(0, GR)], rows_v, sem).wait()

        fire(0, idxA, rowsA, semA)
        fire(1, idxB, rowsB, semB)

        def pair_body(i, _):
            c0 = i * 2
            drain(rowsA, semA)
            reduce_chunk(c0, rowsA)

            @pl.when(c0 + 2 < NCH)
            def _():
                fire(c0 + 2, idxA, rowsA, semA)

            drain(rowsB, semB)
            reduce_chunk(c0 + 1, rowsB)

            @pl.when(c0 + 3 < NCH)
            def _():
                fire(c0 + 3, idxB, rowsB, semB)

            return 0

        lax.fori_loop(0, NCH // 2, pair_body, 0)

    return body(q, idx_flat)


SROWS = 1024


def _stats_body(a_ref, g_ref, g2_ref, out_ref):
    @pl.when(pl.program_id(0) == 0)
    def _():
        out_ref[...] = jnp.zeros_like(out_ref)

    a = a_ref[...]
    g = g_ref[...]
    g2 = g2_ref[...]
    s1 = jnp.sum(jnp.float32(K) * a + g, axis=0, keepdims=True)
    s2 = jnp.sum(jnp.float32(K) * a * a + 2.0 * a * g + g2,
                 axis=0, keepdims=True)
    out_ref[0:1, :] += s1
    out_ref[1:2, :] += s2


def _stats(a, g, g2):
    C = a.shape[1]
    return pl.pallas_call(
        _stats_body,
        grid=(N // SROWS,),
        in_specs=[pl.BlockSpec((SROWS, C), lambda i: (i, 0))] * 3,
        out_specs=pl.BlockSpec((8, C), lambda i: (0, 0)),
        out_shape=jax.ShapeDtypeStruct((8, C), jnp.float32),
        compiler_params=pltpu.CompilerParams(
            dimension_semantics=("arbitrary",)),
    )(a, g, g2)


def _combine_body(a_ref, mx_ref, mn_ref, sums_ref, gm_ref, be_ref, out_ref):
    inv_cnt = jnp.float32(1.0 / (N * K))
    s1 = sums_ref[0:1, :]
    s2 = sums_ref[1:2, :]
    mu = s1 * inv_cnt
    var = s2 * inv_cnt - mu * mu
    s = gm_ref[...] / jnp.sqrt(var + jnp.float32(1e-5))
    a = a_ref[...]
    hmax = (a - mu) * s + be_ref[...] + jnp.maximum(
        s * mx_ref[...], s * mn_ref[...])
    out_ref[...] = jnp.maximum(hmax, 0.0)


def _combine(a, mx, mn, sums, gm2d, be2d):
    C = a.shape[1]
    return pl.pallas_call(
        _combine_body,
        grid=(N // SROWS,),
        in_specs=[
            pl.BlockSpec((SROWS, C), lambda i: (i, 0)),
            pl.BlockSpec((SROWS, C), lambda i: (i, 0)),
            pl.BlockSpec((SROWS, C), lambda i: (i, 0)),
            pl.BlockSpec((8, C), lambda i: (0, 0)),
            pl.BlockSpec((1, C), lambda i: (0, 0)),
            pl.BlockSpec((1, C), lambda i: (0, 0)),
        ],
        out_specs=pl.BlockSpec((SROWS, C), lambda i: (i, 0)),
        out_shape=jax.ShapeDtypeStruct((N, C), jnp.float32),
        compiler_params=pltpu.CompilerParams(
            dimension_semantics=("parallel",)),
    )(a, mx, mn, sums, gm2d, be2d)


def kernel(voxel_coords, voxel_features, W0, b0, g0, be0,
           W1, b1, g1, be1, W2, b2, g2, be2):
    pos = voxel_coords[:, 1:4]
    post = pos.T
    idx = _knn(pos, post)
    idx_flat = idx.reshape(N * K)

    x = voxel_features
    for (W, b, gm, be) in ((W0, b0, g0, be0), (W1, b1, g1, be1),
                           (W2, b2, g2, be2)):
        C = W.shape[1]
        a, q = _mm(x, W, b.reshape(1, C))
        if C < 128:
            qp = jnp.pad(q, ((0, 0), (0, 128 - C)))
            g, g2s, mx, mn = (o[:, :C] for o in
                              _sc_reduce(qp, idx_flat, 128))
        else:
            g, g2s, mx, mn = _sc_reduce(q, idx_flat, C)
        sums = _stats(a, g, g2s)
        x = _combine(a, mx, mn, sums, gm.reshape(1, C), be.reshape(1, C))
    return x


# speedup vs baseline: 1.0474x; 1.0474x over previous
import functools

import jax
import jax.numpy as jnp
from jax import lax
from jax.experimental import pallas as pl
from jax.experimental.pallas import tpu as pltpu
from jax.experimental.pallas import tpu_sc as plsc

N = 8192
K = 16


ROWS = 256
CT = 4096
NT = N // CT
BIG = 1e30


def _knn_body(pos_ref, post_ref, idx_ref, d_ref):
    pos = pos_ref[...]
    px = pos[:, 0:1]
    py = pos[:, 1:2]
    pz = pos[:, 2:3]
    sqi = px * px + py * py + pz * pz

    def bf(v):
        return v.astype(jnp.bfloat16).astype(jnp.float32)

    pxb, pyb, pzb = bf(px), bf(py), bf(pz)
    lt0 = lax.broadcasted_iota(jnp.int32, (1, CT), 1)

    def fill(ct, _):
        col0 = pl.multiple_of(ct * CT, CT)
        tx = post_ref[0:1, pl.ds(col0, CT)]
        ty = post_ref[1:2, pl.ds(col0, CT)]
        tz = post_ref[2:3, pl.ds(col0, CT)]
        sqj = tx * tx + ty * ty + tz * tz
        dot = pxb * bf(tx) + pyb * bf(ty) + pzb * bf(tz)
        d_ref[:, pl.ds(col0, CT)] = sqi + sqj - 2.0 * dot
        return 0

    lax.fori_loop(0, NT, fill, 0)

    am = jnp.full((ROWS, 1), -1, jnp.int32)
    for t in range(K):
        def tile_body(ct, carry, am=am):
            bm, ba = carry
            col0 = pl.multiple_of(ct * CT, CT)
            lt = lt0 + ct * CT
            dt = d_ref[:, pl.ds(col0, CT)]
            dt = jnp.where(lt == am, BIG, dt)
            d_ref[:, pl.ds(col0, CT)] = dt
            tmin = jnp.min(dt, axis=1, keepdims=True)
            targ = jnp.min(jnp.where(dt == tmin, lt, jnp.int32(N)),
                           axis=1, keepdims=True)
            upd = tmin < bm
            return (jnp.where(upd, tmin, bm), jnp.where(upd, targ, ba))

        bm, am = lax.fori_loop(
            0, NT, tile_body,
            (jnp.full((ROWS, 1), BIG, jnp.float32),
             jnp.zeros((ROWS, 1), jnp.int32)))
        idx_ref[:, t:t + 1] = am


def _knn(pos, post):
    return pl.pallas_call(
        _knn_body,
        grid=(N // ROWS,),
        in_specs=[
            pl.BlockSpec((ROWS, 3), lambda i: (i, 0)),
            pl.BlockSpec((3, N), lambda i: (0, 0)),
        ],
        out_specs=pl.BlockSpec((ROWS, K), lambda i: (i, 0)),
        out_shape=jax.ShapeDtypeStruct((N, K), jnp.int32),
        scratch_shapes=[pltpu.VMEM((ROWS, N), jnp.float32)],
        compiler_params=pltpu.CompilerParams(
            dimension_semantics=("arbitrary",)),
    )(pos, post)


MROWS = 1024


def _mm_q_body(x_ref, w_ref, q_ref, *, cin):
    wb = w_ref[cin:2 * cin, :]
    q_ref[...] = jnp.dot(x_ref[...], wb, preferred_element_type=jnp.float32,
                         precision=lax.Precision.HIGHEST)


def _mm_q(x, w):
    cin = w.shape[0] // 2
    cout = w.shape[1]
    return pl.pallas_call(
        functools.partial(_mm_q_body, cin=cin),
        grid=(N // MROWS,),
        in_specs=[
            pl.BlockSpec((MROWS, cin), lambda i: (i, 0)),
            pl.BlockSpec((2 * cin, cout), lambda i: (0, 0)),
        ],
        out_specs=pl.BlockSpec((MROWS, cout), lambda i: (i, 0)),
        out_shape=jax.ShapeDtypeStruct((N, cout), jnp.float32),
        compiler_params=pltpu.CompilerParams(
            dimension_semantics=("parallel",)),
    )(x, w)


def _mm_a_body(x_ref, w_ref, q_ref, b_ref, a_ref, *, cin):
    wa = w_ref[0:cin, :]
    a_ref[...] = (jnp.dot(x_ref[...], wa, preferred_element_type=jnp.float32,
                          precision=lax.Precision.HIGHEST)
                  - q_ref[...] + b_ref[...])


def _mm_a(x, w, q, b2d):
    cin = w.shape[0] // 2
    cout = w.shape[1]
    return pl.pallas_call(
        functools.partial(_mm_a_body, cin=cin),
        grid=(N // MROWS,),
        in_specs=[
            pl.BlockSpec((MROWS, cin), lambda i: (i, 0)),
            pl.BlockSpec((2 * cin, cout), lambda i: (0, 0)),
            pl.BlockSpec((MROWS, cout), lambda i: (i, 0)),
            pl.BlockSpec((1, cout), lambda i: (0, 0)),
        ],
        out_specs=pl.BlockSpec((MROWS, cout), lambda i: (i, 0)),
        out_shape=jax.ShapeDtypeStruct((N, cout), jnp.float32),
        compiler_params=pltpu.CompilerParams(
            dimension_semantics=("parallel",)),
    )(x, w, q, b2d)


def _sc_reduce(q, idx_flat, C):
    mesh = plsc.VectorSubcoreMesh(core_axis_name="c", subcore_axis_name="s")
    NW = 32
    PW = N // NW
    PC = 8
    NCH = PW // PC
    GR = PC * K
    NV = C // 16

    @functools.partial(
        pl.kernel, mesh=mesh,
        out_type=[jax.ShapeDtypeStruct((N, C), jnp.float32)] * 4,
        scratch_types=[
            pltpu.VMEM((GR,), jnp.int32),
            pltpu.VMEM((GR,), jnp.int32),
            pltpu.VMEM((GR, C), jnp.float32),
            pltpu.VMEM((GR, C), jnp.float32),
            pltpu.VMEM((PC, C), jnp.float32),
            pltpu.VMEM((PC, C), jnp.float32),
            pltpu.VMEM((PC, C), jnp.float32),
            pltpu.VMEM((PC, C), jnp.float32),
            pltpu.SemaphoreType.DMA,
            pltpu.SemaphoreType.DMA,
        ],
    )
    def body(q_hbm, idx_hbm, g_hbm, g2_hbm, mx_hbm, mn_hbm,
             idxA, idxB, rowsA, rowsB, o_sum, o_sq, o_mx, o_mn,
             semA, semB):
        wid = lax.axis_index("s") * 2 + lax.axis_index("c")
        base = wid * PW

        def fire(cidx, idx_v, rows_v, sem):
            off = (base + cidx * PC) * K
            pltpu.sync_copy(idx_hbm.at[pl.ds(off, GR)], idx_v)
            return pltpu.async_copy(q_hbm.at[idx_v], rows_v, sem)

        def reduce_chunk(cidx, rows_v):
            def col_body(t, _):
                p = t // NV
                v = t - p * NV
                col = pl.multiple_of(v * 16, 16)
                r0 = p * K
                x0 = rows_v[r0, pl.ds(col, 16)]
                s = x0
                sq = x0 * x0
                mx = x0
                mn = x0
                for j in range(1, K):
                    x = rows_v[r0 + j, pl.ds(col, 16)]
                    s = s + x
                    sq = sq + x * x
                    mx = jnp.maximum(mx, x)
                    mn = jnp.minimum(mn, x)
                o_sum[p, pl.ds(col, 16)] = s
                o_sq[p, pl.ds(col, 16)] = sq
                o_mx[p, pl.ds(col, 16)] = mx
                o_mn[p, pl.ds(col, 16)] = mn
                return 0

            lax.fori_loop(0, PC * NV, col_body, 0)
            row0 = base + cidx * PC
            pltpu.sync_copy(o_sum, g_hbm.at[pl.ds(row0, PC)])
            pltpu.sync_copy(o_sq, g2_hbm.at[pl.ds(row0, PC)])
            pltpu.sync_copy(o_mx, mx_hbm.at[pl.ds(row0, PC)])
            pltpu.sync_copy(o_mn, mn_hbm.at[pl.ds(row0, PC)])

        def drain(rows_v, sem):
            pltpu.make_async_copy(q_hbm.at[pl.ds(0, GR)], rows_v, sem).wait()

        fire(0, idxA, rowsA, semA)
        fire(1, idxB, rowsB, semB)

        def pair_body(i, _):
            c0 = i * 2
            drain(rowsA, semA)
            reduce_chunk(c0, rowsA)

            @pl.when(c0 + 2 < NCH)
            def _():
                fire(c0 + 2, idxA, rowsA, semA)

            drain(rowsB, semB)
            reduce_chunk(c0 + 1, rowsB)

            @pl.when(c0 + 3 < NCH)
            def _():
                fire(c0 + 3, idxB, rowsB, semB)

            return 0

        lax.fori_loop(0, NCH // 2, pair_body, 0)

    return body(q, idx_flat)


SROWS = 1024


def _stats_body(a_ref, g_ref, g2_ref, out_ref):
    @pl.when(pl.program_id(0) == 0)
    def _():
        out_ref[...] = jnp.zeros_like(out_ref)

    a = a_ref[...]
    g = g_ref[...]
    g2 = g2_ref[...]
    s1 = jnp.sum(jnp.float32(K) * a + g, axis=0, keepdims=True)
    s2 = jnp.sum(jnp.float32(K) * a * a + 2.0 * a * g + g2,
                 axis=0, keepdims=True)
    out_ref[0:1, :] += s1
    out_ref[1:2, :] += s2


def _stats(a, g, g2):
    C = a.shape[1]
    return pl.pallas_call(
        _stats_body,
        grid=(N // SROWS,),
        in_specs=[pl.BlockSpec((SROWS, C), lambda i: (i, 0))] * 3,
        out_specs=pl.BlockSpec((8, C), lambda i: (0, 0)),
        out_shape=jax.ShapeDtypeStruct((8, C), jnp.float32),
        compiler_params=pltpu.CompilerParams(
            dimension_semantics=("arbitrary",)),
    )(a, g, g2)


def _combine_body(a_ref, mx_ref, mn_ref, sums_ref, gm_ref, be_ref, out_ref):
    inv_cnt = jnp.float32(1.0 / (N * K))
    s1 = sums_ref[0:1, :]
    s2 = sums_ref[1:2, :]
    mu = s1 * inv_cnt
    var = s2 * inv_cnt - mu * mu
    s = gm_ref[...] / jnp.sqrt(var + jnp.float32(1e-5))
    a = a_ref[...]
    hmax = (a - mu) * s + be_ref[...] + jnp.maximum(
        s * mx_ref[...], s * mn_ref[...])
    out_ref[...] = jnp.maximum(hmax, 0.0)


def _combine(a, mx, mn, sums, gm2d, be2d):
    C = a.shape[1]
    return pl.pallas_call(
        _combine_body,
        grid=(N // SROWS,),
        in_specs=[
            pl.BlockSpec((SROWS, C), lambda i: (i, 0)),
            pl.BlockSpec((SROWS, C), lambda i: (i, 0)),
            pl.BlockSpec((SROWS, C), lambda i: (i, 0)),
            pl.BlockSpec((8, C), lambda i: (0, 0)),
            pl.BlockSpec((1, C), lambda i: (0, 0)),
            pl.BlockSpec((1, C), lambda i: (0, 0)),
        ],
        out_specs=pl.BlockSpec((SROWS, C), lambda i: (i, 0)),
        out_shape=jax.ShapeDtypeStruct((N, C), jnp.float32),
        compiler_params=pltpu.CompilerParams(
            dimension_semantics=("parallel",)),
    )(a, mx, mn, sums, gm2d, be2d)


def kernel(voxel_coords, voxel_features, W0, b0, g0, be0,
           W1, b1, g1, be1, W2, b2, g2, be2):
    pos = voxel_coords[:, 1:4]
    post = pos.T
    idx = _knn(pos, post)
    idx_flat = idx.reshape(N * K)

    x = voxel_features
    for (W, b, gm, be) in ((W0, b0, g0, be0), (W1, b1, g1, be1),
                           (W2, b2, g2, be2)):
        C = W.shape[1]
        q = _mm_q(x, W)
        if C < 128:
            qp = jnp.pad(q, ((0, 0), (0, 128 - C)))
            g, g2s, mx, mn = (o[:, :C] for o in
                              _sc_reduce(qp, idx_flat, 128))
        else:
            g, g2s, mx, mn = _sc_reduce(q, idx_flat, C)
        a = _mm_a(x, W, q, b.reshape(1, C))
        sums = _stats(a, g, g2s)
        x = _combine(a, mx, mn, sums, gm.reshape(1, C), be.reshape(1, C))
    return x
